# baseline (speedup 1.0000x reference)
import jax
import jax.numpy as jnp
from jax import lax
from jax.experimental import pallas as pl
from jax.experimental.pallas import tpu as pltpu

_N_CODES = 1024
_DSEG = 64
_HEADS = 4
_B = 16
_S = 1024
_LOSS_SCALE = 1.25 / (_HEADS * _B * _S * _DSEG)


def _vq_step(z_ref, e_ref, zq_ref, idx_ref, loss_ref,
             e2_ref, enl_ref, id_ref, acc_ref):
    b = pl.program_id(0)
    k = pl.program_id(1)

    @pl.when((b == 0) & (k == 0))
    def _():
        E0 = e_ref[...]
        e2_ref[...] = E0 + E0
        enl_ref[...] = jnp.sum(E0 * E0, axis=1, keepdims=True).reshape(1, _N_CODES)
        di = lax.broadcasted_iota(jnp.int32, (_DSEG, _DSEG), 0)
        dj = lax.broadcasted_iota(jnp.int32, (_DSEG, _DSEG), 1)
        id_ref[...] = (di == dj).astype(jnp.float32)
        acc_ref[0, 0] = 0.0

    zf = z_ref[0, 0].reshape(_DSEG, _S)
    zf_sm = lax.transpose(zf, (1, 0))
    zz = jnp.sum(zf_sm * zf_sm, axis=1, keepdims=True)
    s2 = lax.dot_general(zf, e2_ref[...], (((0,), (1,)), ((), ())),
                         preferred_element_type=jnp.float32)
    d = (zz + enl_ref[...]) - s2
    mind = jnp.min(d, axis=1, keepdims=True)
    iota_l = lax.broadcasted_iota(jnp.int32, (_S, _N_CODES), 1)
    idxs = jnp.min(jnp.where(d == mind, iota_l, _N_CODES),
                   axis=1, keepdims=True)
    idx_l = idxs.reshape(1, _S)
    idx_ref[0, 0] = idx_l
    iota_s = lax.broadcasted_iota(jnp.int32, (_N_CODES, _S), 0)
    oh = (iota_s == idx_l).astype(jnp.bfloat16)
    zq = lax.dot_general(e_ref[...], oh, (((0,), (0,)), ((), ())),
                         preferred_element_type=jnp.float32)
    zq_ref[0, 0] = zq.reshape(_DSEG, 32, 32)

    acc_ref[0, 0] += jnp.sum(mind)

    @pl.when((b == _B - 1) & (k == _HEADS - 1))
    def _():
        loss_ref[0, 0] = acc_ref[0, 0] * _LOSS_SCALE


def kernel(z, emb_weight):
    z5 = z.reshape(_B, _HEADS, _DSEG, 32, 32)
    zq5, idx4, lossv = pl.pallas_call(
        _vq_step,
        grid=(_B, _HEADS),
        in_specs=[
            pl.BlockSpec((1, 1, _DSEG, 32, 32), lambda b, k: (b, k, 0, 0, 0)),
            pl.BlockSpec((_N_CODES, _DSEG), lambda b, k: (0, 0)),
        ],
        out_specs=[
            pl.BlockSpec((1, 1, _DSEG, 32, 32), lambda b, k: (b, k, 0, 0, 0)),
            pl.BlockSpec((1, 1, 1, _S), lambda b, k: (k, b, 0, 0)),
            pl.BlockSpec(block_shape=(1, 1), index_map=lambda b, k: (0, 0),
                         memory_space=pltpu.SMEM),
        ],
        out_shape=[
            jax.ShapeDtypeStruct((_B, _HEADS, _DSEG, 32, 32), jnp.float32),
            jax.ShapeDtypeStruct((_HEADS, _B, 1, _S), jnp.int32),
            jax.ShapeDtypeStruct((1, 1), jnp.float32),
        ],
        scratch_shapes=[
            pltpu.VMEM((_N_CODES, _DSEG), jnp.float32),
            pltpu.VMEM((1, _N_CODES), jnp.float32),
            pltpu.VMEM((_DSEG, _DSEG), jnp.float32),
            pltpu.SMEM((1, 1), jnp.float32),
        ],
        compiler_params=pltpu.CompilerParams(
            dimension_semantics=("arbitrary", "arbitrary"),
        ),
    )(z5, emb_weight)
    z_q = zq5.reshape(z.shape)
    loss = lossv[0, 0]
    min_encoding_indices = idx4.reshape(-1)
    perplexity = jnp.zeros((1,), dtype=jnp.float32)
    cluster_use = jnp.zeros((1,), dtype=jnp.float32)
    return (z_q, loss, perplexity, cluster_use, min_encoding_indices)


# speedup vs baseline: 1.0003x; 1.0003x over previous
import jax
import jax.numpy as jnp
from jax import lax
from jax.experimental import pallas as pl
from jax.experimental.pallas import tpu as pltpu

_N_CODES = 1024
_DSEG = 64
_HEADS = 4
_B = 16
_S = 1024
_LOSS_SCALE = 1.25 / (_HEADS * _B * _S * _DSEG)


def _vq_step(z_ref, e_ref, zq_ref, idx_ref, loss_ref,
             e2_ref, enl_ref, acc_ref):
    b = pl.program_id(0)
    k = pl.program_id(1)

    @pl.when((b == 0) & (k == 0))
    def _():
        E0 = e_ref[...]
        e2_ref[...] = E0 + E0
        enl_ref[...] = jnp.sum(E0 * E0, axis=1, keepdims=True).reshape(1, _N_CODES)
        acc_ref[0, 0] = 0.0

    zf = z_ref[0, 0].reshape(_DSEG, _S)
    zf_sm = lax.transpose(zf, (1, 0))
    zz = jnp.sum(zf_sm * zf_sm, axis=1, keepdims=True)
    s2 = lax.dot_general(zf, e2_ref[...], (((0,), (1,)), ((), ())),
                         preferred_element_type=jnp.float32)
    d = (zz + enl_ref[...]) - s2
    mind = jnp.min(d, axis=1, keepdims=True)
    iota_l = lax.broadcasted_iota(jnp.int32, (_S, _N_CODES), 1)
    idxs = jnp.min(jnp.where(d == mind, iota_l, _N_CODES),
                   axis=1, keepdims=True)
    idx_l = idxs.reshape(1, _S)
    idx_ref[0, 0] = idx_l
    iota_s = lax.broadcasted_iota(jnp.int32, (_N_CODES, _S), 0)
    oh = (iota_s == idx_l).astype(jnp.bfloat16)
    zq = lax.dot_general(e_ref[...], oh, (((0,), (0,)), ((), ())),
                         preferred_element_type=jnp.float32)
    zq_ref[0, 0] = zq.reshape(_DSEG, 32, 32)

    acc_ref[0, 0] += jnp.sum(mind)

    @pl.when((b == _B - 1) & (k == _HEADS - 1))
    def _():
        loss_ref[0, 0] = acc_ref[0, 0] * _LOSS_SCALE


def kernel(z, emb_weight):
    z5 = z.reshape(_B, _HEADS, _DSEG, 32, 32)
    zq5, idx4, lossv = pl.pallas_call(
        _vq_step,
        grid=(_B, _HEADS),
        in_specs=[
            pl.BlockSpec((1, 1, _DSEG, 32, 32), lambda b, k: (b, k, 0, 0, 0)),
            pl.BlockSpec((_N_CODES, _DSEG), lambda b, k: (0, 0)),
        ],
        out_specs=[
            pl.BlockSpec((1, 1, _DSEG, 32, 32), lambda b, k: (b, k, 0, 0, 0)),
            pl.BlockSpec((1, 1, 1, _S), lambda b, k: (k, b, 0, 0)),
            pl.BlockSpec(block_shape=(1, 1), index_map=lambda b, k: (0, 0),
                         memory_space=pltpu.SMEM),
        ],
        out_shape=[
            jax.ShapeDtypeStruct((_B, _HEADS, _DSEG, 32, 32), jnp.float32),
            jax.ShapeDtypeStruct((_HEADS, _B, 1, _S), jnp.int32),
            jax.ShapeDtypeStruct((1, 1), jnp.float32),
        ],
        scratch_shapes=[
            pltpu.VMEM((_N_CODES, _DSEG), jnp.float32),
            pltpu.VMEM((1, _N_CODES), jnp.float32),
            pltpu.SMEM((1, 1), jnp.float32),
        ],
        compiler_params=pltpu.CompilerParams(
            dimension_semantics=("arbitrary", "arbitrary"),
        ),
    )(z5, emb_weight)
    z_q = zq5.reshape(z.shape)
    loss = lossv[0, 0]
    min_encoding_indices = idx4.reshape(-1)
    perplexity = jnp.zeros((1,), dtype=jnp.float32)
    cluster_use = jnp.zeros((1,), dtype=jnp.float32)
    return (z_q, loss, perplexity, cluster_use, min_encoding_indices)


# speedup vs baseline: 1.0230x; 1.0227x over previous
import jax
import jax.numpy as jnp
from jax import lax
from jax.experimental import pallas as pl
from jax.experimental.pallas import tpu as pltpu

_N_CODES = 1024
_DSEG = 64
_HEADS = 4
_B = 16
_S = 1024
_LOSS_SCALE = 1.25 / (_HEADS * _B * _S * _DSEG)


def _vq_step(z_ref, e_ref, zq_ref, idx_ref, loss_ref,
             e2_ref, enl_ref, acc_ref):
    b = pl.program_id(0)
    k = pl.program_id(1)

    @pl.when((b == 0) & (k == 0))
    def _():
        E0 = e_ref[...]
        e2_ref[...] = E0 + E0
        enl_ref[...] = jnp.sum(E0 * E0, axis=1, keepdims=True).reshape(1, _N_CODES)
        acc_ref[0, 0] = 0.0

    iota_l = lax.broadcasted_iota(jnp.int32, (_S, _N_CODES), 1)
    iota_s = lax.broadcasted_iota(jnp.int32, (_N_CODES, _S), 0)
    minds = []
    for j in range(2):
        zf = z_ref[0, j].reshape(_DSEG, _S)
        zf_sm = lax.transpose(zf, (1, 0))
        zz = jnp.sum(zf_sm * zf_sm, axis=1, keepdims=True)
        s2 = lax.dot_general(zf, e2_ref[...], (((0,), (1,)), ((), ())),
                             preferred_element_type=jnp.float32)
        d = (zz + enl_ref[...]) - s2
        mind = jnp.min(d, axis=1, keepdims=True)
        idxs = jnp.min(jnp.where(d == mind, iota_l, _N_CODES),
                       axis=1, keepdims=True)
        idx_l = idxs.reshape(1, _S)
        idx_ref[j, 0] = idx_l
        oh = (iota_s == idx_l).astype(jnp.bfloat16)
        zq = lax.dot_general(e_ref[...], oh, (((0,), (0,)), ((), ())),
                             preferred_element_type=jnp.float32)
        zq_ref[0, j] = zq.reshape(_DSEG, 32, 32)
        minds.append(mind)

    acc_ref[0, 0] += jnp.sum(minds[0]) + jnp.sum(minds[1])

    @pl.when((b == _B - 1) & (k == 1))
    def _():
        loss_ref[0, 0] = acc_ref[0, 0] * _LOSS_SCALE


def kernel(z, emb_weight):
    z5 = z.reshape(_B, _HEADS, _DSEG, 32, 32)
    zq5, idx4, lossv = pl.pallas_call(
        _vq_step,
        grid=(_B, _HEADS // 2),
        in_specs=[
            pl.BlockSpec((1, 2, _DSEG, 32, 32), lambda b, k: (b, k, 0, 0, 0)),
            pl.BlockSpec((_N_CODES, _DSEG), lambda b, k: (0, 0)),
        ],
        out_specs=[
            pl.BlockSpec((1, 2, _DSEG, 32, 32), lambda b, k: (b, k, 0, 0, 0)),
            pl.BlockSpec((2, 1, 1, _S), lambda b, k: (k, b, 0, 0)),
            pl.BlockSpec(block_shape=(1, 1), index_map=lambda b, k: (0, 0),
                         memory_space=pltpu.SMEM),
        ],
        out_shape=[
            jax.ShapeDtypeStruct((_B, _HEADS, _DSEG, 32, 32), jnp.float32),
            jax.ShapeDtypeStruct((_HEADS, _B, 1, _S), jnp.int32),
            jax.ShapeDtypeStruct((1, 1), jnp.float32),
        ],
        scratch_shapes=[
            pltpu.VMEM((_N_CODES, _DSEG), jnp.float32),
            pltpu.VMEM((1, _N_CODES), jnp.float32),
            pltpu.SMEM((1, 1), jnp.float32),
        ],
        compiler_params=pltpu.CompilerParams(
            dimension_semantics=("arbitrary", "arbitrary"),
        ),
    )(z5, emb_weight)
    z_q = zq5.reshape(z.shape)
    loss = lossv[0, 0]
    min_encoding_indices = idx4.reshape(-1)
    perplexity = jnp.zeros((1,), dtype=jnp.float32)
    cluster_use = jnp.zeros((1,), dtype=jnp.float32)
    return (z_q, loss, perplexity, cluster_use, min_encoding_indices)


# speedup vs baseline: 1.0295x; 1.0064x over previous
import jax
import jax.numpy as jnp
from jax import lax
from jax.experimental import pallas as pl
from jax.experimental.pallas import tpu as pltpu

_N_CODES = 1024
_DSEG = 64
_HEADS = 4
_B = 16
_S = 1024
_LOSS_SCALE = 1.25 / (_HEADS * _B * _S * _DSEG)


def _vq_step(z_ref, e_ref, zq_ref, idx_ref, loss_ref,
             e2_ref, enl_ref, acc_ref):
    b = pl.program_id(0)
    k = pl.program_id(1)

    @pl.when((b == 0) & (k == 0))
    def _():
        E0 = e_ref[...]
        e2_ref[...] = E0 + E0
        enl_ref[...] = jnp.sum(E0 * E0, axis=1, keepdims=True).reshape(1, _N_CODES)
        acc_ref[0, 0] = 0.0

    iota_l = lax.broadcasted_iota(jnp.int32, (_S, _N_CODES), 1)
    iota_s = lax.broadcasted_iota(jnp.int32, (_N_CODES, _S), 0)
    minds = []
    for j in range(_HEADS):
        zf = z_ref[0, j].reshape(_DSEG, _S)
        zf_sm = lax.transpose(zf, (1, 0))
        zz = jnp.sum(zf_sm * zf_sm, axis=1, keepdims=True)
        s2 = lax.dot_general(zf, e2_ref[...], (((0,), (1,)), ((), ())),
                             preferred_element_type=jnp.float32)
        d = (zz + enl_ref[...]) - s2
        mind = jnp.min(d, axis=1, keepdims=True)
        idxs = jnp.min(jnp.where(d == mind, iota_l, _N_CODES),
                       axis=1, keepdims=True)
        idx_l = idxs.reshape(1, _S)
        idx_ref[j, 0] = idx_l
        oh = (iota_s == idx_l).astype(jnp.bfloat16)
        zq = lax.dot_general(e_ref[...], oh, (((0,), (0,)), ((), ())),
                             preferred_element_type=jnp.float32)
        zq_ref[0, j] = zq.reshape(_DSEG, 32, 32)
        minds.append(mind)

    acc_ref[0, 0] += sum(jnp.sum(m) for m in minds)

    @pl.when(b == _B - 1)
    def _():
        loss_ref[0, 0] = acc_ref[0, 0] * _LOSS_SCALE


def kernel(z, emb_weight):
    z5 = z.reshape(_B, _HEADS, _DSEG, 32, 32)
    zq5, idx4, lossv = pl.pallas_call(
        _vq_step,
        grid=(_B, 1),
        in_specs=[
            pl.BlockSpec((1, _HEADS, _DSEG, 32, 32), lambda b, k: (b, 0, 0, 0, 0)),
            pl.BlockSpec((_N_CODES, _DSEG), lambda b, k: (0, 0)),
        ],
        out_specs=[
            pl.BlockSpec((1, _HEADS, _DSEG, 32, 32), lambda b, k: (b, 0, 0, 0, 0)),
            pl.BlockSpec((_HEADS, 1, 1, _S), lambda b, k: (0, b, 0, 0)),
            pl.BlockSpec(block_shape=(1, 1), index_map=lambda b, k: (0, 0),
                         memory_space=pltpu.SMEM),
        ],
        out_shape=[
            jax.ShapeDtypeStruct((_B, _HEADS, _DSEG, 32, 32), jnp.float32),
            jax.ShapeDtypeStruct((_HEADS, _B, 1, _S), jnp.int32),
            jax.ShapeDtypeStruct((1, 1), jnp.float32),
        ],
        scratch_shapes=[
            pltpu.VMEM((_N_CODES, _DSEG), jnp.float32),
            pltpu.VMEM((1, _N_CODES), jnp.float32),
            pltpu.SMEM((1, 1), jnp.float32),
        ],
        compiler_params=pltpu.CompilerParams(
            dimension_semantics=("arbitrary", "arbitrary"),
        ),
    )(z5, emb_weight)
    z_q = zq5.reshape(z.shape)
    loss = lossv[0, 0]
    min_encoding_indices = idx4.reshape(-1)
    perplexity = jnp.zeros((1,), dtype=jnp.float32)
    cluster_use = jnp.zeros((1,), dtype=jnp.float32)
    return (z_q, loss, perplexity, cluster_use, min_encoding_indices)
